# baseline (speedup 1.0000x reference)
import functools

import jax
import jax.numpy as jnp
from jax import lax
from jax.experimental import pallas as pl
from jax.experimental.pallas import tpu as pltpu
from jax.experimental.pallas import tpu_sc as plsc

NUM_EMB = 100000
DIM = 64

_info = plsc.get_sparse_core_info()
_NC, _NS = _info.num_cores, _info.num_subcores
_NW = _NC * _NS


@functools.partial(jax.jit, static_argnames=("b_per_w", "chunk"))
def _gather_sc(idx_flat, w, *, b_per_w, chunk):
    nchunk = b_per_w // chunk
    mesh = plsc.VectorSubcoreMesh(core_axis_name="c", subcore_axis_name="s")

    @functools.partial(
        pl.kernel,
        mesh=mesh,
        out_type=jax.ShapeDtypeStruct((b_per_w * _NW, DIM), jnp.float32),
        scratch_types=[
            pltpu.VMEM((b_per_w,), jnp.int32),
            pltpu.VMEM((chunk, DIM), jnp.float32),
            pltpu.VMEM((chunk, DIM), jnp.float32),
            pltpu.SemaphoreType.DMA,
            pltpu.SemaphoreType.DMA,
            pltpu.SemaphoreType.DMA,
            pltpu.SemaphoreType.DMA,
        ],
        compiler_params=pltpu.CompilerParams(use_tc_tiling_on_sc=False),
    )
    def k(idx_hbm, table_hbm, out_hbm, idx_v, rows0, rows1, g0, g1, o0, o1):
        wid = lax.axis_index("s") * _NC + lax.axis_index("c")
        base = wid * b_per_w
        pltpu.sync_copy(idx_hbm.at[pl.ds(base, b_per_w)], idx_v)

        bufs = (rows0, rows1)
        gsems = (g0, g1)
        osems = (o0, o1)

        def start_gather(c, buf, sem):
            return pltpu.async_copy(
                table_hbm.at[idx_v.at[pl.ds(c * chunk, chunk)]], buf, sem
            )

        def start_out(c, buf, sem):
            return pltpu.async_copy(
                buf, out_hbm.at[pl.ds(base + c * chunk, chunk)], sem
            )

        gathers = [None, None]
        outs = [None, None]
        gathers[0] = start_gather(0, bufs[0], gsems[0])
        for c in range(nchunk):
            b = c & 1
            gathers[b].wait()
            nxt = c + 1
            if nxt < nchunk:
                nb = nxt & 1
                if outs[nb] is not None:
                    outs[nb].wait()
                gathers[nb] = start_gather(nxt, bufs[nb], gsems[nb])
            outs[b] = start_out(c, bufs[b], osems[b])
        for o in outs:
            if o is not None:
                o.wait()

    return k(idx_flat, w)


def kernel(token_ids, w):
    n_tok = token_ids.shape[0] * token_ids.shape[1]
    idx_flat = token_ids.reshape(n_tok).astype(jnp.int32)
    b_per_w = n_tok // _NW
    out = _gather_sc(idx_flat, w, b_per_w=b_per_w, chunk=800)
    return out.reshape(token_ids.shape + (DIM,))
